# baseline (speedup 1.0000x reference)
import jax
import jax.numpy as jnp
from jax.experimental import pallas as pl
from jax.experimental.pallas import tpu as pltpu

F32 = jnp.float32
HI = jax.lax.Precision.HIGHEST


def _mdot(a, b, precision=None):
    return jax.lax.dot_general(a, b, (((1,), (0,)), ((), ())),
                               preferred_element_type=F32, precision=precision)


def _mdot_tb(a, b, precision=None):
    return jax.lax.dot_general(a, b, (((1,), (1,)), ((), ())),
                               preferred_element_type=F32, precision=precision)


def _bdot(a, b):
    return jax.lax.dot_general(a.astype(jnp.bfloat16), b.astype(jnp.bfloat16),
                               (((1,), (0,)), ((), ())),
                               preferred_element_type=F32)


def _bdot_tb(a, b):
    return jax.lax.dot_general(a.astype(jnp.bfloat16), b.astype(jnp.bfloat16),
                               (((1,), (1,)), ((), ())),
                               preferred_element_type=F32)


def _mdot3(a, w3_ref):
    return (_mdot(a, w3_ref[0]) + _mdot(a, w3_ref[1])) + _mdot(a, w3_ref[2])


def _pad_hw(x3):
    H, W, C = x3.shape
    zr = jnp.zeros((1, W, C), F32)
    x3 = jnp.concatenate([zr, x3, zr], axis=0)
    zc = jnp.zeros((H + 2, 1, C), F32)
    return jnp.concatenate([zc, x3, zc], axis=1)


_S2TAP = [(1, -1), (0, 0), (1, 0), (0, 1)]
_TTAPS = {0: [(0, -1), (2, 0)], 1: [(1, 0), (3, 1)]}


def _conv3x3_chunked(src_val, pp_ref, wmat_ref, C, HW=64, nchunk=4):
    padded = _pad_hw(src_val.reshape(HW, HW, C))
    rows = HW // nchunk
    m = rows * HW
    outs = []
    for ck in range(nchunk):
        for t, (dy, dx) in enumerate([(dy, dx) for dy in (0, 1, 2)
                                      for dx in (0, 1, 2)]):
            slab = padded[dy + ck * rows: dy + ck * rows + rows,
                          dx: dx + HW, :].reshape(m, C)
            pp_ref[0:m, t * C:(t + 1) * C] = slab
        outs.append(_mdot3(pp_ref[0:m, 0:9 * C], wmat_ref))
    return jnp.concatenate(outs, axis=0)


def _resblock_inkernel(h, pp_ref, wa_ref, wb_ref, C):
    t = _conv3x3_chunked(jnp.maximum(h, 0.0), pp_ref, wa_ref, C)
    t = _mdot3(jnp.maximum(t, 0.0), wb_ref)
    return h + t


def _enc_kernel(p1_ref, texth_ref, w1_ref, w2_ref, w3_ref,
                r1a_ref, r1b_ref, r2a_ref, r2b_ref,
                wq_ref, wk_ref, wv_ref, wo_ref,
                ba_ref, bb_ref, bc_ref, bd_ref,
                sr_ref, pl1_ref, pp_ref):
    for pp in range(4):
        pl1_ref[pp] = jnp.maximum(_mdot3(p1_ref[0, pp], w1_ref), 0.0)
    h2_parts = []
    rows = 16
    for ck in range(4):
        for t in range(16):
            kh, kw = t // 4, t % 4
            p, di = _S2TAP[kh]
            q, dj = _S2TAP[kw]
            plane = _pad_hw(pl1_ref[2 * p + q].reshape(64, 64, 128))
            slab = plane[1 + di + ck * rows: 1 + di + ck * rows + rows,
                         1 + dj: 1 + dj + 64, :].reshape(rows * 64, 128)
            pp_ref[:, t * 128:(t + 1) * 128] = slab
        h2_parts.append(jnp.maximum(_mdot3(pp_ref[...], w2_ref), 0.0))
    h2 = jnp.concatenate(h2_parts, axis=0)
    s = _conv3x3_chunked(h2, pp_ref, w3_ref, 128)
    s = _resblock_inkernel(s, pp_ref, r1a_ref, r1b_ref, 64)
    s = _resblock_inkernel(s, pp_ref, r2a_ref, r2b_ref, 64)
    k = _bdot(texth_ref[0], wk_ref[...])
    v = _bdot(texth_ref[0], wv_ref[...])
    q = _bdot(s, wq_ref[...])
    sc = _bdot_tb(q, k) * 0.125
    m = jnp.max(sc, axis=1, keepdims=True)
    e = jnp.exp(sc - m)
    a = e / jnp.sum(e, axis=1, keepdims=True)
    tok = s + _bdot(_bdot(a, v), wo_ref[...])
    tok = _resblock_inkernel(tok, pp_ref, ba_ref, bb_ref, 64)
    tok = _resblock_inkernel(tok, pp_ref, bc_ref, bd_ref, 64)
    sr_ref[0] = tok


def _quant_kernel(sr_ref, e_ref, esq_ref, qf_ref, cnt_ref):
    flat = sr_ref[0]
    mm = _mdot_tb(flat, e_ref[...])
    d2 = (jnp.sum(flat * flat, axis=1, keepdims=True)
          - 2.0 * mm + esq_ref[...])
    dmin = jnp.min(d2, axis=1, keepdims=True)
    iota = jax.lax.broadcasted_iota(jnp.int32, (4096, 512), 1)
    idxm = jnp.min(jnp.where(d2 == dmin, iota, 512), axis=1, keepdims=True)
    onehot = jnp.where(iota == idxm, 1.0, 0.0).astype(F32)
    qf_ref[0] = _mdot(onehot, e_ref[...], precision=HI)
    cnt_ref[0] = jnp.sum(onehot, axis=0, keepdims=True)


def _perp_kernel(cnt_ref, out_ref):
    counts = jnp.sum(cnt_ref[...], axis=0)
    probs = counts * (1.0 / 131072.0)
    h = -jnp.sum(probs * jnp.log(probs + 1e-10), axis=1, keepdims=True)
    out_ref[...] = jnp.exp(h)


def _zero_border(buf):
    H = buf.shape[0]
    buf[0:1, :, :] = jnp.zeros((1,) + buf.shape[1:], F32)
    buf[H - 1:H, :, :] = jnp.zeros((1,) + buf.shape[1:], F32)
    buf[:, 0:1, :] = jnp.zeros((buf.shape[0], 1, buf.shape[2]), F32)
    buf[:, H - 1:H, :] = jnp.zeros((buf.shape[0], 1, buf.shape[2]), F32)


def _conv3x3_ref(src_pad, dst_pad, pp_ref, wmat_ref, Cin, Cout,
                 relu_in=False, skip_pad=None):
    for ck in range(4):
        r0 = ck * 16
        for t, (dy, dx) in enumerate([(dy, dx) for dy in (0, 1, 2)
                                      for dx in (0, 1, 2)]):
            slab = src_pad[dy + r0: dy + r0 + 16, dx: dx + 64, 0:Cin]
            if relu_in:
                slab = jnp.maximum(slab, 0.0)
            pp_ref[0:1024, t * Cin:(t + 1) * Cin] = slab.reshape(1024, Cin)
        out = _mdot3(pp_ref[0:1024, 0:9 * Cin], wmat_ref)
        dst_pad[1 + r0: 17 + r0, 1:65, 0:Cout] = out.reshape(16, 64, Cout)


def _dec_kernel(qf_ref, w1_ref, r1a_ref, r1b_ref, r2a_ref, r2b_ref,
                t1_ref, out_ref, pp_ref, padA, padB):
    _zero_border(padA)
    _zero_border(padB)
    padA[1:65, 1:65, 0:64] = qf_ref[0].reshape(64, 64, 64)
    _conv3x3_ref(padA, padB, pp_ref, w1_ref, 64, 128)
    for ra, rb in ((r1a_ref, r1b_ref), (r2a_ref, r2b_ref)):
        _conv3x3_ref(padB, padA, pp_ref, ra, 128, 128, relu_in=True)
        t = _mdot3(jnp.maximum(padA[1:65, 1:65, :].reshape(4096, 128), 0.0), rb)
        padB[1:65, 1:65, :] = (padB[1:65, 1:65, :].reshape(4096, 128)
                               + t).reshape(64, 64, 128)
    for pp in range(4):
        p, q = pp // 2, pp % 2
        for ck in range(4):
            r0 = ck * 16
            for t, (kh, di) in enumerate(_TTAPS[p]):
                for u, (kw, dj) in enumerate(_TTAPS[q]):
                    s = t * 2 + u
                    slab = padB[1 + di + r0: 17 + di + r0, 1 + dj: 65 + dj, :]
                    pp_ref[0:1024, s * 128:(s + 1) * 128] = slab.reshape(1024, 128)
            out_ref[0, pp, ck * 1024:(ck + 1) * 1024, :] = jnp.maximum(
                _mdot3(pp_ref[0:1024, 0:512], t1_ref.at[pp]), 0.0)


def _dect2_kernel(h_ref, t2_ref, out_ref, pp_ref, pad_ref):
    _zero_border(pad_ref)
    pad_ref[1:129, 1:129, :] = h_ref[0]
    padded = pad_ref
    for ck in range(16):
        for pp in range(4):
            p, q = pp // 2, pp % 2
            for t, (kh, di) in enumerate(_TTAPS[p]):
                for u, (kw, dj) in enumerate(_TTAPS[q]):
                    s = t * 2 + u
                    slab = padded[1 + di + ck * 8: 9 + di + ck * 8,
                                  1 + dj: 129 + dj, :]
                    pp_ref[:, pp * 256 + s * 64: pp * 256 + (s + 1) * 64] = (
                        slab.reshape(1024, 64))
        out_ref[0, ck * 1024:(ck + 1) * 1024, :] = _mdot3(
            pp_ref[...], t2_ref)


def _wmat3x3(w):
    return w.transpose(2, 3, 1, 0).reshape(9 * w.shape[1], w.shape[0])


def _shift2(x, di, dj):
    B, H, W, C = x.shape
    xp = jnp.pad(x, ((0, 0), (1, 1), (1, 1), (0, 0)))
    return xp[:, 1 + di:1 + di + H, 1 + dj:1 + dj + W, :]


def kernel(imgh, texth, text_mask, p):
    B = imgh.shape[0]
    cp = pltpu.CompilerParams(dimension_semantics=("arbitrary",),
                              vmem_limit_bytes=52 * 1024 * 1024)
    def _s3(w):
        w1 = w.astype(jnp.bfloat16).astype(F32)
        r1 = w - w1
        w2 = r1.astype(jnp.bfloat16).astype(F32)
        w3 = jnp.zeros_like(w1)
        return jnp.stack([w1, w2, w3], axis=0)

    x = imgh.transpose(0, 2, 3, 1)
    planes = [[x[:, pp::2, qq::2, :] for qq in range(2)] for pp in range(2)]
    slabs = []
    for kh in range(4):
        pr, di = _S2TAP[kh]
        for kw in range(4):
            qr, dj = _S2TAP[kw]
            slabs.append(_shift2(planes[pr][qr], di, dj))
    pat1 = jnp.stack(slabs, axis=3).reshape(B, 128, 128, 48)
    pat1 = pat1.reshape(B, 128, 128, 48)
    pat1 = jnp.stack([pat1[:, pr::2, qr::2, :].reshape(B, 4096, 48)
                      for pr in range(2) for qr in range(2)], axis=1)
    w1m = _s3(p['enc_w1'].transpose(2, 3, 1, 0).reshape(48, 128))
    w2m = _s3(p['enc_w2'].transpose(2, 3, 1, 0).reshape(2048, 128))
    w3m = _s3(_wmat3x3(p['enc_w3']))
    r1am, r2am = _s3(_wmat3x3(p['enc_r1a'])), _s3(_wmat3x3(p['enc_r2a']))
    r1bm, r2bm = _s3(p['enc_r1b'][:, :, 0, 0].T), _s3(p['enc_r2b'][:, :, 0, 0].T)
    bam, bcm = _s3(_wmat3x3(p['reb_r1a'])), _s3(_wmat3x3(p['reb_r2a']))
    bbm, bdm = _s3(p['reb_r1b'][:, :, 0, 0].T), _s3(p['reb_r2b'][:, :, 0, 0].T)
    dw1m = _s3(_wmat3x3(p['dec_w1']))
    dr1am, dr2am = _s3(_wmat3x3(p['dec_r1a'])), _s3(_wmat3x3(p['dec_r2a']))
    dr1bm, dr2bm = (_s3(p['dec_r1b'][:, :, 0, 0].T),
                    _s3(p['dec_r2b'][:, :, 0, 0].T))
    def _tmats(w):
        mats = []
        for pr in range(2):
            for qr in range(2):
                rows = jnp.stack([w[:, :, kh, kw]
                                  for kh, _ in _TTAPS[pr]
                                  for kw, _ in _TTAPS[qr]], axis=2)
                mats.append(rows.transpose(2, 1, 0).reshape(-1, w.shape[0]))
        return jnp.stack(mats, axis=0)
    t1m = jax.vmap(_s3)(_tmats(p['dec_t1']))
    t2m4 = _tmats(p['dec_t2'])
    t2m0 = jnp.zeros((1024, 12), F32)
    for _pp in range(4):
        t2m0 = t2m0.at[_pp * 256:(_pp + 1) * 256,
                       _pp * 3:(_pp + 1) * 3].set(t2m4[_pp])
    t2m = _s3(t2m0)
    E = p['codebook']
    esq = jnp.sum(E * E, axis=1)[None, :]

    bcast = lambda *dims: pl.BlockSpec(dims, lambda b: (0,) * len(dims))
    perb = lambda *dims: pl.BlockSpec(dims, lambda b: (b,) + (0,) * (len(dims) - 1))

    sr = pl.pallas_call(
        _enc_kernel,
        grid=(B,),
        in_specs=[perb(1, 4, 4096, 48), perb(1, 64, 256),
                  bcast(3, 48, 128), bcast(3, 2048, 128), bcast(3, 1152, 64),
                  bcast(3, 576, 64), bcast(3, 64, 64), bcast(3, 576, 64),
                  bcast(3, 64, 64),
                  bcast(64, 64), bcast(256, 64), bcast(256, 64), bcast(64, 64),
                  bcast(3, 576, 64), bcast(3, 64, 64), bcast(3, 576, 64),
                  bcast(3, 64, 64)],
        out_specs=perb(1, 4096, 64),
        out_shape=jax.ShapeDtypeStruct((B, 4096, 64), F32),
        scratch_shapes=[pltpu.VMEM((4, 4096, 128), F32),
                        pltpu.VMEM((1024, 2048), F32)],
        compiler_params=pltpu.CompilerParams(
            dimension_semantics=("arbitrary",),
            vmem_limit_bytes=56 * 1024 * 1024),
        name="enc_attn",
    )(pat1, texth, w1m, w2m, w3m, r1am, r1bm, r2am, r2bm,
      p['wq'], p['wk'], p['wv'], p['wo'], bam, bbm, bcm, bdm)

    qf, cnt = pl.pallas_call(
        _quant_kernel,
        grid=(B,),
        in_specs=[perb(1, 4096, 64), bcast(512, 64), bcast(1, 512)],
        out_specs=[perb(1, 4096, 64), perb(1, 1, 512)],
        out_shape=[jax.ShapeDtypeStruct((B, 4096, 64), F32),
                   jax.ShapeDtypeStruct((B, 1, 512), F32)],
        compiler_params=cp,
        name="quantize",
    )(sr, E, esq)

    perp = pl.pallas_call(
        _perp_kernel,
        in_specs=[pl.BlockSpec((B, 1, 512), lambda: (0, 0, 0))],
        out_specs=pl.BlockSpec((1, 1), lambda: (0, 0)),
        out_shape=jax.ShapeDtypeStruct((1, 1), F32),
        grid=(),
        compiler_params=pltpu.CompilerParams(),
        name="perplexity",
    )(cnt.reshape(B, 1, 512))

    dec1 = pl.pallas_call(
        _dec_kernel,
        grid=(B,),
        in_specs=[perb(1, 4096, 64),
                  bcast(3, 576, 128), bcast(3, 1152, 128), bcast(3, 128, 128),
                  bcast(3, 1152, 128), bcast(3, 128, 128),
                  bcast(4, 3, 512, 64)],
        out_specs=perb(1, 4, 4096, 64),
        out_shape=jax.ShapeDtypeStruct((B, 4, 4096, 64), F32),
        scratch_shapes=[pltpu.VMEM((1024, 1152), F32),
                        pltpu.VMEM((66, 66, 128), F32),
                        pltpu.VMEM((66, 66, 128), F32)],
        compiler_params=pltpu.CompilerParams(
            dimension_semantics=("arbitrary",),
            vmem_limit_bytes=56 * 1024 * 1024),
        name="decoder",
    )(qf, dw1m, dr1am, dr1bm, dr2am, dr2bm, t1m)

    d = dec1.reshape(B, 2, 2, 64, 64, 64).transpose(0, 3, 1, 4, 2, 5)
    h128 = d.reshape(B, 128, 128, 64)

    out12 = pl.pallas_call(
        _dect2_kernel,
        grid=(B,),
        in_specs=[perb(1, 128, 128, 64), bcast(3, 1024, 12)],
        out_specs=perb(1, 16384, 12),
        out_shape=jax.ShapeDtypeStruct((B, 16384, 12), F32),
        scratch_shapes=[pltpu.VMEM((1024, 1024), F32),
                        pltpu.VMEM((130, 130, 64), F32)],
        compiler_params=pltpu.CompilerParams(
            dimension_semantics=("arbitrary",),
            vmem_limit_bytes=56 * 1024 * 1024),
        name="dec_t2",
    )(h128, t2m)

    xr = out12.reshape(B, 128, 128, 2, 2, 3)
    x_recon = xr.transpose(0, 5, 1, 3, 2, 4).reshape(B, 3, 256, 256)
    return x_recon, perp.reshape(())


# speedup vs baseline: 1.1610x; 1.1610x over previous
import jax
import jax.numpy as jnp
from jax.experimental import pallas as pl
from jax.experimental.pallas import tpu as pltpu

F32 = jnp.float32
HI = jax.lax.Precision.HIGHEST


def _mdot(a, b, precision=None):
    return jax.lax.dot_general(a, b, (((1,), (0,)), ((), ())),
                               preferred_element_type=F32, precision=precision)


def _mdot_tb(a, b, precision=None):
    return jax.lax.dot_general(a, b, (((1,), (1,)), ((), ())),
                               preferred_element_type=F32, precision=precision)


def _bdot(a, b):
    return jax.lax.dot_general(a.astype(jnp.bfloat16), b.astype(jnp.bfloat16),
                               (((1,), (0,)), ((), ())),
                               preferred_element_type=F32)


def _bdot_tb(a, b):
    return jax.lax.dot_general(a.astype(jnp.bfloat16), b.astype(jnp.bfloat16),
                               (((1,), (1,)), ((), ())),
                               preferred_element_type=F32)


def _mdot3(a, w3_ref):
    return _mdot(a, w3_ref[0]) + _mdot(a, w3_ref[1])


def _pad_hw(x3):
    H, W, C = x3.shape
    zr = jnp.zeros((1, W, C), F32)
    x3 = jnp.concatenate([zr, x3, zr], axis=0)
    zc = jnp.zeros((H + 2, 1, C), F32)
    return jnp.concatenate([zc, x3, zc], axis=1)


_S2TAP = [(1, -1), (0, 0), (1, 0), (0, 1)]
_TTAPS = {0: [(0, -1), (2, 0)], 1: [(1, 0), (3, 1)]}


def _conv3x3_chunked(src_val, pp_ref, wmat_ref, C, HW=64, nchunk=8):
    padded = _pad_hw(src_val.reshape(HW, HW, C))
    rows = HW // nchunk
    m = rows * HW
    outs = []
    for ck in range(nchunk):
        for t, (dy, dx) in enumerate([(dy, dx) for dy in (0, 1, 2)
                                      for dx in (0, 1, 2)]):
            slab = padded[dy + ck * rows: dy + ck * rows + rows,
                          dx: dx + HW, :].reshape(m, C)
            pp_ref[0:m, t * C:(t + 1) * C] = slab
        outs.append(_mdot3(pp_ref[0:m, 0:9 * C], wmat_ref))
    return jnp.concatenate(outs, axis=0)


def _resblock_inkernel(h, pp_ref, wa_ref, wb_ref, C):
    t = _conv3x3_chunked(jnp.maximum(h, 0.0), pp_ref, wa_ref, C)
    t = _mdot3(jnp.maximum(t, 0.0), wb_ref)
    return h + t


def _enc_kernel(p1_ref, texth_ref, w1_ref, w2_ref, w3_ref,
                r1a_ref, r1b_ref, r2a_ref, r2b_ref,
                wq_ref, wk_ref, wv_ref, wo_ref,
                ba_ref, bb_ref, bc_ref, bd_ref,
                sr_ref, pl1_ref, pp_ref):
    for pp in range(4):
        pl1_ref[pp] = jnp.maximum(_mdot3(p1_ref[0, pp], w1_ref), 0.0)
    h2_parts = []
    rows = 8
    for ck in range(8):
        for t in range(16):
            kh, kw = t // 4, t % 4
            p, di = _S2TAP[kh]
            q, dj = _S2TAP[kw]
            plane = _pad_hw(pl1_ref[2 * p + q].reshape(64, 64, 128))
            slab = plane[1 + di + ck * rows: 1 + di + ck * rows + rows,
                         1 + dj: 1 + dj + 64, :].reshape(rows * 64, 128)
            pp_ref[:, t * 128:(t + 1) * 128] = slab
        h2_parts.append(jnp.maximum(_mdot3(pp_ref[...], w2_ref), 0.0))
    h2 = jnp.concatenate(h2_parts, axis=0)
    s = _conv3x3_chunked(h2, pp_ref, w3_ref, 128)
    s = _resblock_inkernel(s, pp_ref, r1a_ref, r1b_ref, 64)
    s = _resblock_inkernel(s, pp_ref, r2a_ref, r2b_ref, 64)
    k = _bdot(texth_ref[0], wk_ref[...])
    v = _bdot(texth_ref[0], wv_ref[...])
    q = _bdot(s, wq_ref[...])
    sc = _bdot_tb(q, k) * 0.125
    m = jnp.max(sc, axis=1, keepdims=True)
    e = jnp.exp(sc - m)
    a = e / jnp.sum(e, axis=1, keepdims=True)
    tok = s + _bdot(_bdot(a, v), wo_ref[...])
    tok = _resblock_inkernel(tok, pp_ref, ba_ref, bb_ref, 64)
    tok = _resblock_inkernel(tok, pp_ref, bc_ref, bd_ref, 64)
    sr_ref[0] = tok


def _quant_kernel(sr_ref, e_ref, esq_ref, qf_ref, cnt_ref):
    flat = sr_ref[0]
    mm = _mdot_tb(flat, e_ref[...])
    d2 = (jnp.sum(flat * flat, axis=1, keepdims=True)
          - 2.0 * mm + esq_ref[...])
    dmin = jnp.min(d2, axis=1, keepdims=True)
    iota = jax.lax.broadcasted_iota(jnp.int32, (4096, 512), 1)
    idxm = jnp.min(jnp.where(d2 == dmin, iota, 512), axis=1, keepdims=True)
    onehot = jnp.where(iota == idxm, 1.0, 0.0).astype(F32)
    qf_ref[0] = _mdot(onehot, e_ref[...], precision=HI)
    cnt_ref[0] = jnp.sum(onehot, axis=0, keepdims=True)


def _perp_kernel(cnt_ref, out_ref):
    counts = jnp.sum(cnt_ref[...], axis=0)
    probs = counts * (1.0 / 131072.0)
    h = -jnp.sum(probs * jnp.log(probs + 1e-10), axis=1, keepdims=True)
    out_ref[...] = jnp.exp(h)


def _zero_border(buf):
    H = buf.shape[0]
    buf[0:1, :, :] = jnp.zeros((1,) + buf.shape[1:], F32)
    buf[H - 1:H, :, :] = jnp.zeros((1,) + buf.shape[1:], F32)
    buf[:, 0:1, :] = jnp.zeros((buf.shape[0], 1, buf.shape[2]), F32)
    buf[:, H - 1:H, :] = jnp.zeros((buf.shape[0], 1, buf.shape[2]), F32)


def _conv3x3_ref(src_pad, dst_pad, pp_ref, wmat_ref, Cin, Cout,
                 relu_in=False, skip_pad=None):
    for ck in range(4):
        r0 = ck * 16
        for t, (dy, dx) in enumerate([(dy, dx) for dy in (0, 1, 2)
                                      for dx in (0, 1, 2)]):
            slab = src_pad[dy + r0: dy + r0 + 16, dx: dx + 64, 0:Cin]
            if relu_in:
                slab = jnp.maximum(slab, 0.0)
            pp_ref[0:1024, t * Cin:(t + 1) * Cin] = slab.reshape(1024, Cin)
        out = _mdot3(pp_ref[0:1024, 0:9 * Cin], wmat_ref)
        dst_pad[1 + r0: 17 + r0, 1:65, 0:Cout] = out.reshape(16, 64, Cout)


def _dec_kernel(qf_ref, w1_ref, r1a_ref, r1b_ref, r2a_ref, r2b_ref,
                t1_ref, out_ref, pp_ref, padA, padB):
    _zero_border(padA)
    _zero_border(padB)
    padA[1:65, 1:65, 0:64] = qf_ref[0].reshape(64, 64, 64)
    _conv3x3_ref(padA, padB, pp_ref, w1_ref, 64, 128)
    for ra, rb in ((r1a_ref, r1b_ref), (r2a_ref, r2b_ref)):
        _conv3x3_ref(padB, padA, pp_ref, ra, 128, 128, relu_in=True)
        t = _mdot3(jnp.maximum(padA[1:65, 1:65, :].reshape(4096, 128), 0.0), rb)
        padB[1:65, 1:65, :] = (padB[1:65, 1:65, :].reshape(4096, 128)
                               + t).reshape(64, 64, 128)
    for pp in range(4):
        p, q = pp // 2, pp % 2
        for ck in range(4):
            r0 = ck * 16
            for t, (kh, di) in enumerate(_TTAPS[p]):
                for u, (kw, dj) in enumerate(_TTAPS[q]):
                    s = t * 2 + u
                    slab = padB[1 + di + r0: 17 + di + r0, 1 + dj: 65 + dj, :]
                    pp_ref[0:1024, s * 128:(s + 1) * 128] = slab.reshape(1024, 128)
            out_ref[0, pp, ck * 1024:(ck + 1) * 1024, :] = jnp.maximum(
                _mdot3(pp_ref[0:1024, 0:512], t1_ref.at[pp]), 0.0)


def _dect2_kernel(h_ref, t2_ref, out_ref, pp_ref, pad_ref):
    _zero_border(pad_ref)
    pad_ref[1:129, 1:129, :] = h_ref[0]
    padded = pad_ref
    for ck in range(16):
        for pp in range(4):
            p, q = pp // 2, pp % 2
            for t, (kh, di) in enumerate(_TTAPS[p]):
                for u, (kw, dj) in enumerate(_TTAPS[q]):
                    s = t * 2 + u
                    slab = padded[1 + di + ck * 8: 9 + di + ck * 8,
                                  1 + dj: 129 + dj, :]
                    pp_ref[:, pp * 256 + s * 64: pp * 256 + (s + 1) * 64] = (
                        slab.reshape(1024, 64))
        out_ref[0, ck * 1024:(ck + 1) * 1024, :] = _mdot3(
            pp_ref[...], t2_ref)


def _wmat3x3(w):
    return w.transpose(2, 3, 1, 0).reshape(9 * w.shape[1], w.shape[0])


def _shift2(x, di, dj):
    B, H, W, C = x.shape
    xp = jnp.pad(x, ((0, 0), (1, 1), (1, 1), (0, 0)))
    return xp[:, 1 + di:1 + di + H, 1 + dj:1 + dj + W, :]


def kernel(imgh, texth, text_mask, p):
    B = imgh.shape[0]
    cp = pltpu.CompilerParams(dimension_semantics=("arbitrary",),
                              vmem_limit_bytes=52 * 1024 * 1024)
    def _s3(w):
        w1 = w.astype(jnp.bfloat16).astype(F32)
        r1 = w - w1
        w2 = r1.astype(jnp.bfloat16).astype(F32)
        return jnp.stack([w1, w2], axis=0)

    x = imgh.transpose(0, 2, 3, 1)
    planes = [[x[:, pp::2, qq::2, :] for qq in range(2)] for pp in range(2)]
    slabs = []
    for kh in range(4):
        pr, di = _S2TAP[kh]
        for kw in range(4):
            qr, dj = _S2TAP[kw]
            slabs.append(_shift2(planes[pr][qr], di, dj))
    pat1 = jnp.stack(slabs, axis=3).reshape(B, 128, 128, 48)
    pat1 = pat1.reshape(B, 128, 128, 48)
    pat1 = jnp.stack([pat1[:, pr::2, qr::2, :].reshape(B, 4096, 48)
                      for pr in range(2) for qr in range(2)], axis=1)
    w1m = _s3(p['enc_w1'].transpose(2, 3, 1, 0).reshape(48, 128))
    w2m = _s3(p['enc_w2'].transpose(2, 3, 1, 0).reshape(2048, 128))
    w3m = _s3(_wmat3x3(p['enc_w3']))
    r1am, r2am = _s3(_wmat3x3(p['enc_r1a'])), _s3(_wmat3x3(p['enc_r2a']))
    r1bm, r2bm = _s3(p['enc_r1b'][:, :, 0, 0].T), _s3(p['enc_r2b'][:, :, 0, 0].T)
    bam, bcm = _s3(_wmat3x3(p['reb_r1a'])), _s3(_wmat3x3(p['reb_r2a']))
    bbm, bdm = _s3(p['reb_r1b'][:, :, 0, 0].T), _s3(p['reb_r2b'][:, :, 0, 0].T)
    dw1m = _s3(_wmat3x3(p['dec_w1']))
    dr1am, dr2am = _s3(_wmat3x3(p['dec_r1a'])), _s3(_wmat3x3(p['dec_r2a']))
    dr1bm, dr2bm = (_s3(p['dec_r1b'][:, :, 0, 0].T),
                    _s3(p['dec_r2b'][:, :, 0, 0].T))
    def _tmats(w):
        mats = []
        for pr in range(2):
            for qr in range(2):
                rows = jnp.stack([w[:, :, kh, kw]
                                  for kh, _ in _TTAPS[pr]
                                  for kw, _ in _TTAPS[qr]], axis=2)
                mats.append(rows.transpose(2, 1, 0).reshape(-1, w.shape[0]))
        return jnp.stack(mats, axis=0)
    t1m = jax.vmap(_s3)(_tmats(p['dec_t1']))
    t2m4 = _tmats(p['dec_t2'])
    t2m0 = jnp.zeros((1024, 12), F32)
    for _pp in range(4):
        t2m0 = t2m0.at[_pp * 256:(_pp + 1) * 256,
                       _pp * 3:(_pp + 1) * 3].set(t2m4[_pp])
    t2m = _s3(t2m0)
    E = p['codebook']
    esq = jnp.sum(E * E, axis=1)[None, :]

    bcast = lambda *dims: pl.BlockSpec(dims, lambda b: (0,) * len(dims))
    perb = lambda *dims: pl.BlockSpec(dims, lambda b: (b,) + (0,) * (len(dims) - 1))

    sr = pl.pallas_call(
        _enc_kernel,
        grid=(B,),
        in_specs=[perb(1, 4, 4096, 48), perb(1, 64, 256),
                  bcast(2, 48, 128), bcast(2, 2048, 128), bcast(2, 1152, 64),
                  bcast(2, 576, 64), bcast(2, 64, 64), bcast(2, 576, 64),
                  bcast(2, 64, 64),
                  bcast(64, 64), bcast(256, 64), bcast(256, 64), bcast(64, 64),
                  bcast(2, 576, 64), bcast(2, 64, 64), bcast(2, 576, 64),
                  bcast(2, 64, 64)],
        out_specs=perb(1, 4096, 64),
        out_shape=jax.ShapeDtypeStruct((B, 4096, 64), F32),
        scratch_shapes=[pltpu.VMEM((4, 4096, 128), F32),
                        pltpu.VMEM((512, 2048), F32)],
        compiler_params=pltpu.CompilerParams(
            dimension_semantics=("arbitrary",),
            vmem_limit_bytes=57 * 1024 * 1024),
        name="enc_attn",
    )(pat1, texth, w1m, w2m, w3m, r1am, r1bm, r2am, r2bm,
      p['wq'], p['wk'], p['wv'], p['wo'], bam, bbm, bcm, bdm)

    qf, cnt = pl.pallas_call(
        _quant_kernel,
        grid=(B,),
        in_specs=[perb(1, 4096, 64), bcast(512, 64), bcast(1, 512)],
        out_specs=[perb(1, 4096, 64), perb(1, 1, 512)],
        out_shape=[jax.ShapeDtypeStruct((B, 4096, 64), F32),
                   jax.ShapeDtypeStruct((B, 1, 512), F32)],
        compiler_params=cp,
        name="quantize",
    )(sr, E, esq)

    perp = pl.pallas_call(
        _perp_kernel,
        in_specs=[pl.BlockSpec((B, 1, 512), lambda: (0, 0, 0))],
        out_specs=pl.BlockSpec((1, 1), lambda: (0, 0)),
        out_shape=jax.ShapeDtypeStruct((1, 1), F32),
        grid=(),
        compiler_params=pltpu.CompilerParams(),
        name="perplexity",
    )(cnt.reshape(B, 1, 512))

    dec1 = pl.pallas_call(
        _dec_kernel,
        grid=(B,),
        in_specs=[perb(1, 4096, 64),
                  bcast(2, 576, 128), bcast(2, 1152, 128), bcast(2, 128, 128),
                  bcast(2, 1152, 128), bcast(2, 128, 128),
                  bcast(4, 2, 512, 64)],
        out_specs=perb(1, 4, 4096, 64),
        out_shape=jax.ShapeDtypeStruct((B, 4, 4096, 64), F32),
        scratch_shapes=[pltpu.VMEM((1024, 1152), F32),
                        pltpu.VMEM((66, 66, 128), F32),
                        pltpu.VMEM((66, 66, 128), F32)],
        compiler_params=pltpu.CompilerParams(
            dimension_semantics=("arbitrary",),
            vmem_limit_bytes=56 * 1024 * 1024),
        name="decoder",
    )(qf, dw1m, dr1am, dr1bm, dr2am, dr2bm, t1m)

    d = dec1.reshape(B, 2, 2, 64, 64, 64).transpose(0, 3, 1, 4, 2, 5)
    h128 = d.reshape(B, 128, 128, 64)

    out12 = pl.pallas_call(
        _dect2_kernel,
        grid=(B,),
        in_specs=[perb(1, 128, 128, 64), bcast(2, 1024, 12)],
        out_specs=perb(1, 16384, 12),
        out_shape=jax.ShapeDtypeStruct((B, 16384, 12), F32),
        scratch_shapes=[pltpu.VMEM((1024, 1024), F32),
                        pltpu.VMEM((130, 130, 64), F32)],
        compiler_params=pltpu.CompilerParams(
            dimension_semantics=("arbitrary",),
            vmem_limit_bytes=56 * 1024 * 1024),
        name="dec_t2",
    )(h128, t2m)

    xr = out12.reshape(B, 128, 128, 2, 2, 3)
    x_recon = xr.transpose(0, 5, 1, 3, 2, 4).reshape(B, 3, 256, 256)
    return x_recon, perp.reshape(())
